# Initial kernel scaffold; baseline (speedup 1.0000x reference)
import jax
import jax.numpy as jnp
from jax.experimental import pallas as pl


def kernel(traffic, packets, capacity, scale, le_w1, le_b1, le_w2, le_b2, pe_w1, pe_b1, pe_w2, pe_b2, am_w1, am_b1, am_w2, am_b2, am_w3, am_b3, ro_w1, ro_b1, ro_w2, ro_b2, ro_w3, ro_b3, lg_wi, lg_wr, lg_bi, lg_br, pg_wi, pg_wr, pg_bi, pg_br, link_to_path, path_ids, sequence_path, sequence_links, n_links, n_paths):
    raise NotImplementedError("write your pallas kernel here")


# speedup vs baseline: 2.3161x; 2.3161x over previous
import functools

import jax
import jax.numpy as jnp
from jax import lax
from jax.experimental import pallas as pl
from jax.experimental.pallas import tpu as pltpu
from jax.experimental.pallas import tpu_sc as plsc

NP_ = 100000
NL = 20000
D = 64
NW = 32
PT = 3136
NPP = NW * PT
NV = 128
RNG = 160
LPAD = NV * RNG
PCAP = 102400
CHUNK = 512
RCH = 128
DW = 128
NEG = -3.402823e38
POS = 3.402823e38

_mesh = plsc.VectorSubcoreMesh(core_axis_name="c", subcore_axis_name="s")


def _wid():
    return lax.axis_index("s") * 2 + lax.axis_index("c")


def _iota16():
    return lax.broadcasted_iota(jnp.int32, (16,), 0)


def _ds8(off, n):
    return pl.ds(pl.multiple_of(off, 8), n)


@functools.partial(
    pl.kernel,
    out_type=(
        jax.ShapeDtypeStruct((NV * PCAP,), jnp.int32),
        jax.ShapeDtypeStruct((NV * 16,), jnp.int32),
        jax.ShapeDtypeStruct((LPAD,), jnp.float32),
    ),
    mesh=_mesh,
    compiler_params=pltpu.CompilerParams(needs_layout_passes=False),
    scratch_types=[
        pltpu.VMEM((2000,), jnp.int32),
        pltpu.VMEM((4096,), jnp.int32),
        pltpu.VMEM((RNG,), jnp.int32),
        pltpu.VMEM((RNG,), jnp.float32),
        pltpu.VMEM((16,), jnp.int32),
    ],
)
def _sc_plan(sl_hbm, plan_hbm, nv_hbm, cnt_hbm, slv, pbuf, cacc, cf, nbuf):
    wid = _wid()
    iota = _iota16()
    ones = jnp.ones((16,), jnp.int32)
    for voff in (0, 32, 64, 96):
        v = wid + voff
        lo = v * RNG

        def zero_cnt(j, _):
            cacc[pl.ds(j * 16, 16)] = jnp.zeros((16,), jnp.int32)
            return 0

        lax.fori_loop(0, RNG // 16, zero_cnt, 0)

        def chunk(c, carry):
            ptr0, hb0 = carry
            pltpu.sync_copy(sl_hbm.at[_ds8(c * 2000, 2000)], slv)

            def step(i, ptr):
                sl = slv[pl.ds(i * 16, 16)]
                m = (sl >= lo) & (sl < lo + RNG)
                ll = jnp.clip(sl - lo, 0, RNG - 1)
                pid = c * 2000 + i * 16 + iota
                packed = (pid << 9) | ll
                mi32 = m.astype(jnp.int32)
                pos = ptr + plsc.cumsum(mi32) - 1
                plsc.store_scatter(pbuf, [pos], packed, mask=m)
                plsc.addupdate_scatter(cacc, [ll], ones, mask=m)
                return ptr + jnp.sum(mi32)

            ptr = lax.fori_loop(0, 125, step, ptr0)

            def flush(args):
                p, hb = args
                pltpu.sync_copy(pbuf.at[pl.ds(0, 2048)],
                                plan_hbm.at[_ds8(v * PCAP + hb, 2048)])

                def mv(i, _):
                    pbuf[pl.ds(i * 16, 16)] = pbuf[pl.ds(2048 + i * 16, 16)]
                    return 0

                lax.fori_loop(0, 128, mv, 0)
                return (p - 2048, hb + 2048)

            return lax.cond(ptr >= 2048, flush, lambda a: a, (ptr, hb0))

        ptr, hbase = lax.fori_loop(0, 50, chunk,
                                   (jnp.int32(0), jnp.int32(0)))

        def ztail(i, _):
            pbuf[pl.ds(ptr + i * 16, 16)] = jnp.full((16,), RNG, jnp.int32)
            return 0

        lax.fori_loop(0, 32, ztail, 0)
        pltpu.sync_copy(pbuf.at[pl.ds(0, 2048)],
                        plan_hbm.at[_ds8(v * PCAP + hbase, 2048)])
        pltpu.sync_copy(pbuf.at[pl.ds(2048, 512)],
                        plan_hbm.at[_ds8(v * PCAP + hbase + 2048, 512)])
        nbuf[...] = jnp.where(iota == 0, ptr + hbase, 0)
        pltpu.sync_copy(nbuf, nv_hbm.at[_ds8(v * 16, 16)])

        def cvt(j, _):
            cf[pl.ds(j * 16, 16)] = cacc[pl.ds(j * 16, 16)].astype(jnp.float32)
            return 0

        lax.fori_loop(0, RNG // 16, cvt, 0)
        pltpu.sync_copy(cf, cnt_hbm.at[_ds8(lo, RNG)])


@functools.partial(
    pl.kernel,
    out_type=jax.ShapeDtypeStruct((NPP, DW), jnp.float32),
    mesh=_mesh,
    compiler_params=pltpu.CompilerParams(needs_layout_passes=False),
    scratch_types=[
        pltpu.VMEM((CHUNK,), jnp.int32),
        pltpu.VMEM((CHUNK, DW), jnp.float32),
        pltpu.VMEM((64,), jnp.int32),
        pltpu.VMEM((64, DW), jnp.float32),
        pltpu.SemaphoreType.DMA,
    ],
)
def _sc_gather(idx_hbm, tbl_hbm, out_hbm, idxv, rows, idxv2, rows2, sem):
    base = _wid() * PT
    for off in (0, 512, 1024, 1536, 2048, 2560):
        pltpu.sync_copy(idx_hbm.at[_ds8(base + off, CHUNK)], idxv)
        pltpu.async_copy(tbl_hbm.at[idxv], rows, sem).wait()
        pltpu.sync_copy(rows, out_hbm.at[_ds8(base + off, CHUNK)])
    pltpu.sync_copy(idx_hbm.at[_ds8(base + 3072, 64)], idxv2)
    pltpu.async_copy(tbl_hbm.at[idxv2], rows2, sem).wait()
    pltpu.sync_copy(rows2, out_hbm.at[_ds8(base + 3072, 64)])


@functools.partial(
    pl.kernel,
    out_type=(
        jax.ShapeDtypeStruct((LPAD, D), jnp.float32),
        jax.ShapeDtypeStruct((LPAD, D), jnp.float32),
        jax.ShapeDtypeStruct((LPAD, D), jnp.float32),
    ),
    mesh=_mesh,
    compiler_params=pltpu.CompilerParams(needs_layout_passes=False),
    scratch_types=[
        pltpu.VMEM((RCH,), jnp.int32),
        pltpu.VMEM((RCH,), jnp.int32),
        pltpu.VMEM((RCH, DW), jnp.float32),
        pltpu.VMEM((RNG + 1, D), jnp.float32),
        pltpu.VMEM((RNG + 1, D), jnp.float32),
        pltpu.VMEM((RNG + 1, D), jnp.float32),
        pltpu.VMEM((16,), jnp.int32),
        pltpu.SemaphoreType.DMA,
    ],
)
def _sc_reduce(h_hbm, plan_hbm, nv_hbm, sum_hbm, max_hbm, min_hbm,
               pk, pidx, rows, asum, amax, amin, nbuf, sem):
    wid = _wid()
    iota = _iota16()
    cols = [iota + r * 16 for r in range(4)]
    zf = jnp.zeros((16,), jnp.float32)
    negf = jnp.full((16,), NEG, jnp.float32)
    posf = jnp.full((16,), POS, jnp.float32)
    for voff in (0, 32, 64, 96):
        v = wid + voff
        lo = v * RNG
        pltpu.sync_copy(nv_hbm.at[_ds8(v * 16, 16)], nbuf)
        n = nbuf[...][0]

        def zrow(l, _):
            rowl = jnp.full((16,), l, jnp.int32)
            for r in range(4):
                plsc.store_scatter(asum, [rowl, cols[r]], zf)
                plsc.store_scatter(amax, [rowl, cols[r]], negf)
                plsc.store_scatter(amin, [rowl, cols[r]], posf)
            return 0

        lax.fori_loop(0, RNG + 1, zrow, 0)

        def chunk(c, _):
            pltpu.sync_copy(plan_hbm.at[_ds8(v * PCAP + c * RCH, RCH)], pk)

            def unp(i, _):
                pidx[pl.ds(i * 16, 16)] = pk[pl.ds(i * 16, 16)] >> 9
                return 0

            lax.fori_loop(0, RCH // 16, unp, 0)
            pltpu.async_copy(h_hbm.at[pidx], rows, sem).wait()

            def rmw(i, _):
                pkv = pk[pl.ds(i * 16, 16)]
                lv = pkv & 511
                for lane in range(16):
                    l = lv[lane]
                    rowj = jnp.full((16,), i * 16 + lane, jnp.int32)
                    rowl = jnp.full((16,), l, jnp.int32)
                    for r in range(4):
                        hv = plsc.load_gather(rows, [rowj, cols[r]])
                        plsc.addupdate_scatter(asum, [rowl, cols[r]], hv)
                        mv = plsc.load_gather(amax, [rowl, cols[r]])
                        plsc.store_scatter(amax, [rowl, cols[r]],
                                           jnp.maximum(mv, hv))
                        nv_ = plsc.load_gather(amin, [rowl, cols[r]])
                        plsc.store_scatter(amin, [rowl, cols[r]],
                                           jnp.minimum(nv_, hv))
                return 0

            lax.fori_loop(0, RCH // 16, rmw, 0)
            return 0

        lax.fori_loop(0, (n + RCH - 1) // RCH, chunk, 0)
        pltpu.sync_copy(asum.at[pl.ds(0, RNG)], sum_hbm.at[_ds8(lo, RNG)])
        pltpu.sync_copy(amax.at[pl.ds(0, RNG)], max_hbm.at[_ds8(lo, RNG)])
        pltpu.sync_copy(amin.at[pl.ds(0, RNG)], min_hbm.at[_ds8(lo, RNG)])


@functools.partial(
    pl.kernel,
    out_type=jax.ShapeDtypeStruct((LPAD,), jnp.float32),
    mesh=_mesh,
    compiler_params=pltpu.CompilerParams(needs_layout_passes=False),
    scratch_types=[
        pltpu.VMEM((LPAD // NW, 8), jnp.float32),
        pltpu.VMEM((LPAD // NW,), jnp.float32),
        pltpu.VMEM((LPAD // NW,), jnp.float32),
        pltpu.VMEM((LPAD // NW,), jnp.float32),
    ],
)
def _sc_delay(y_hbm, cap_hbm, scl_hbm, out_hbm, yv, cv, sv, dbuf):
    per = LPAD // NW
    lo = _wid() * per
    iota = _iota16()
    zc = jnp.zeros((16,), jnp.int32)
    pltpu.sync_copy(y_hbm.at[_ds8(lo, per)], yv)
    pltpu.sync_copy(cap_hbm.at[_ds8(lo, per)], cv)
    pltpu.sync_copy(scl_hbm.at[_ds8(lo, per)], sv)

    def step(j, _):
        rows = j * 16 + iota
        y0 = plsc.load_gather(yv, [rows, zc])
        occ = 1.0 / (1.0 + jnp.exp(-y0))
        c = cv[pl.ds(j * 16, 16)]
        s = sv[pl.ds(j * 16, 16)]
        rc = (c * 24631.01 + 21166.35) * (s * 5.77 + 10.5)
        qd = occ * 32000.0 / rc
        qd = jnp.where(qd != qd, 0.0, qd)
        td = 1000.0 / rc
        td = jnp.where(jnp.abs(td) == jnp.inf, 0.0, td)
        dbuf[pl.ds(j * 16, 16)] = qd + td
        return 0

    lax.fori_loop(0, per // 16, step, 0)
    pltpu.sync_copy(dbuf, out_hbm.at[_ds8(lo, per)])


@functools.partial(
    pl.kernel,
    out_type=jax.ShapeDtypeStruct((NPP,), jnp.float32),
    mesh=_mesh,
    compiler_params=pltpu.CompilerParams(needs_layout_passes=False),
    scratch_types=[
        pltpu.VMEM((LPAD,), jnp.float32),
        pltpu.VMEM((CHUNK,), jnp.int32),
        pltpu.VMEM((CHUNK,), jnp.float32),
        pltpu.VMEM((64,), jnp.int32),
        pltpu.VMEM((64,), jnp.float32),
    ],
)
def _sc_out_gather(dl_hbm, idx_hbm, out_hbm, tbl, idxv, obuf, idxv2, obuf2):
    base = _wid() * PT
    pltpu.sync_copy(dl_hbm, tbl)
    for off in (0, 512, 1024, 1536, 2048, 2560):
        pltpu.sync_copy(idx_hbm.at[_ds8(base + off, CHUNK)], idxv)

        def step(i, _):
            av = idxv[pl.ds(i * 16, 16)]
            obuf[pl.ds(i * 16, 16)] = plsc.load_gather(tbl, [av])
            return 0

        lax.fori_loop(0, CHUNK // 16, step, 0)
        pltpu.sync_copy(obuf, out_hbm.at[_ds8(base + off, CHUNK)])
    pltpu.sync_copy(idx_hbm.at[_ds8(base + 3072, 64)], idxv2)

    def step2(i, _):
        av = idxv2[pl.ds(i * 16, 16)]
        obuf2[pl.ds(i * 16, 16)] = plsc.load_gather(tbl, [av])
        return 0

    lax.fori_loop(0, 4, step2, 0)
    pltpu.sync_copy(obuf2, out_hbm.at[_ds8(base + 3072, 64)])


def _full(shape):
    return pl.BlockSpec(shape, lambda i: tuple(0 for _ in shape))


def _tc_link_embed_body(cs, w1, b1, w2, b2, out):
    t = jnp.maximum(jnp.dot(cs[...], w1[...],
                            preferred_element_type=jnp.float32) + b1[...], 0.0)
    h = jnp.maximum(jnp.dot(t, w2[...],
                            preferred_element_type=jnp.float32) + b2[...], 0.0)
    out[...] = jnp.concatenate(
        [h, jnp.zeros((h.shape[0], DW - D), jnp.float32)], axis=1)


def _tc_link_embed(cs, w1, b1, w2, b2):
    blk = 2000
    return pl.pallas_call(
        _tc_link_embed_body,
        grid=(NL // blk,),
        in_specs=[
            pl.BlockSpec((blk, 2), lambda i: (i, 0)),
            _full(w1.shape), _full(b1.shape), _full(w2.shape), _full(b2.shape),
        ],
        out_specs=pl.BlockSpec((blk, DW), lambda i: (i, 0)),
        out_shape=jax.ShapeDtypeStruct((NL, DW), jnp.float32),
    )(cs, w1, b1, w2, b2)


def _gru_math(mx, mh, h):
    xz, xr, xh = mx[:, :D], mx[:, D:2 * D], mx[:, 2 * D:]
    hz, hr, hh = mh[:, :D], mh[:, D:2 * D], mh[:, 2 * D:]
    z = jax.nn.sigmoid(xz + hz)
    r = jax.nn.sigmoid(xr + hr)
    cand = jnp.tanh(xh + r * hh)
    return z * h + (1.0 - z) * cand


def _tc_pgru_body(tp, g, pw1, pb1, pw2, pb2, wi, wr, bi, br, out):
    t = jnp.maximum(jnp.dot(tp[...], pw1[...],
                            preferred_element_type=jnp.float32) + pb1[...], 0.0)
    h0 = jnp.maximum(jnp.dot(t, pw2[...],
                             preferred_element_type=jnp.float32) + pb2[...], 0.0)
    gv = g[...][:, :D]
    mx = jnp.dot(gv, wi[...], preferred_element_type=jnp.float32) + bi[...]
    mh = jnp.dot(h0, wr[...], preferred_element_type=jnp.float32) + br[...]
    h = _gru_math(mx, mh, h0)
    nz = jnp.sum((gv != 0.0).astype(jnp.float32), axis=1, keepdims=True)
    h = jnp.where(nz > 0.0, h, h0)
    out[...] = jnp.concatenate(
        [h, jnp.zeros((h.shape[0], DW - D), jnp.float32)], axis=1)


def _tc_pgru(tp, g, pw1, pb1, pw2, pb2, wi, wr, bi, br):
    blk = 2000
    return pl.pallas_call(
        _tc_pgru_body,
        grid=(NP_ // blk,),
        in_specs=[
            pl.BlockSpec((blk, 2), lambda i: (i, 0)),
            pl.BlockSpec((blk, DW), lambda i: (i, 0)),
            _full(pw1.shape), _full(pb1.shape),
            _full(pw2.shape), _full(pb2.shape),
            _full(wi.shape), _full(wr.shape),
            _full(bi.shape), _full(br.shape),
        ],
        out_specs=pl.BlockSpec((blk, DW), lambda i: (i, 0)),
        out_shape=jax.ShapeDtypeStruct((NP_, DW), jnp.float32),
    )(tp, g, pw1, pb1, pw2, pb2, wi, wr, bi, br)


def _tc_link_body(s, mx_, mn_, cnt, ls, w1, b1, w2, b2, w3, b3,
                  wi, wr, bi, br, out):
    sv = s[...]
    cntv = cnt[...]
    has = cntv > 0.0
    mean = jnp.where(has, sv / jnp.maximum(cntv, 1.0), 0.0)
    mxv = jnp.where(has, mx_[...], 0.0)
    mnv = jnp.where(has, mn_[...], 0.0)
    mi = jnp.concatenate([sv, mean, mxv, mnv], axis=1)
    pa = jnp.maximum(jnp.dot(mi, w1[...],
                             preferred_element_type=jnp.float32) + b1[...], 0.0)
    pa = jnp.maximum(jnp.dot(pa, w2[...],
                             preferred_element_type=jnp.float32) + b2[...], 0.0)
    pa = jnp.maximum(jnp.dot(pa, w3[...],
                             preferred_element_type=jnp.float32) + b3[...], 0.0)
    h = ls[...][:, :D]
    mxg = jnp.dot(pa, wi[...], preferred_element_type=jnp.float32) + bi[...]
    mhg = jnp.dot(h, wr[...], preferred_element_type=jnp.float32) + br[...]
    hn = _gru_math(mxg, mhg, h)
    out[...] = jnp.concatenate(
        [hn, jnp.zeros((hn.shape[0], DW - D), jnp.float32)], axis=1)


def _tc_link(s, mx_, mn_, cnt2d, ls, w1, b1, w2, b2, w3, b3, wi, wr, bi, br):
    blk = 2000
    return pl.pallas_call(
        _tc_link_body,
        grid=(NL // blk,),
        in_specs=[
            pl.BlockSpec((blk, D), lambda i: (i, 0)),
            pl.BlockSpec((blk, D), lambda i: (i, 0)),
            pl.BlockSpec((blk, D), lambda i: (i, 0)),
            pl.BlockSpec((blk, 1), lambda i: (i, 0)),
            pl.BlockSpec((blk, DW), lambda i: (i, 0)),
            _full(w1.shape), _full(b1.shape), _full(w2.shape), _full(b2.shape),
            _full(w3.shape), _full(b3.shape),
            _full(wi.shape), _full(wr.shape), _full(bi.shape), _full(br.shape),
        ],
        out_specs=pl.BlockSpec((blk, DW), lambda i: (i, 0)),
        out_shape=jax.ShapeDtypeStruct((NL, DW), jnp.float32),
    )(s, mx_, mn_, cnt2d, ls, w1, b1, w2, b2, w3, b3, wi, wr, bi, br)


def _tc_readout_body(ls, w1, b1, w2, b2, w3p, b3p, out):
    o = jnp.maximum(jnp.dot(ls[...][:, :D], w1[...],
                            preferred_element_type=jnp.float32) + b1[...], 0.0)
    o = jnp.maximum(jnp.dot(o, w2[...],
                            preferred_element_type=jnp.float32) + b2[...], 0.0)
    out[...] = jnp.dot(o, w3p[...],
                       preferred_element_type=jnp.float32) + b3p[...]


def _tc_readout(ls, w1, b1, w2, b2, w3p, b3p):
    blk = 2000
    return pl.pallas_call(
        _tc_readout_body,
        grid=(NL // blk,),
        in_specs=[
            pl.BlockSpec((blk, DW), lambda i: (i, 0)),
            _full(w1.shape), _full(b1.shape), _full(w2.shape), _full(b2.shape),
            _full(w3p.shape), _full(b3p.shape),
        ],
        out_specs=pl.BlockSpec((blk, 8), lambda i: (i, 0)),
        out_shape=jax.ShapeDtypeStruct((LPAD, 8), jnp.float32),
    )(ls, w1, b1, w2, b2, w3p, b3p)


def kernel(traffic, packets, capacity, scale, le_w1, le_b1, le_w2, le_b2,
           pe_w1, pe_b1, pe_w2, pe_b2, am_w1, am_b1, am_w2, am_b2, am_w3,
           am_b3, ro_w1, ro_b1, ro_w2, ro_b2, ro_w3, ro_b3, lg_wi, lg_wr,
           lg_bi, lg_br, pg_wi, pg_wr, pg_bi, pg_br, link_to_path, path_ids,
           sequence_path, sequence_links, n_links, n_paths):
    cs = jnp.stack([capacity, scale], axis=1)
    tp = jnp.stack([traffic, packets], axis=1)
    a_pad = jnp.pad(link_to_path, (0, NPP - NP_))
    cap_pad = jnp.pad(capacity, (0, LPAD - NL))
    scl_pad = jnp.pad(scale, (0, LPAD - NL))
    w3p = jnp.pad(ro_w3, ((0, 0), (0, 7)))
    b3p = jnp.pad(ro_b3, (0, 7))

    ls = _tc_link_embed(cs, le_w1, le_b1, le_w2, le_b2)
    plan, nva, counts = _sc_plan(sequence_links)
    cnt2d = counts[:NL].reshape(NL, 1)

    for _ in range(8):
        g = _sc_gather(a_pad, ls)
        h = _tc_pgru(tp, g, pe_w1, pe_b1, pe_w2, pe_b2,
                     pg_wi, pg_wr, pg_bi, pg_br)
        s, mx_, mn_ = _sc_reduce(h, plan, nva)
        ls = _tc_link(s, mx_, mn_, cnt2d, ls,
                      am_w1, am_b1, am_w2, am_b2, am_w3, am_b3,
                      lg_wi, lg_wr, lg_bi, lg_br)

    y = _tc_readout(ls, ro_w1, ro_b1, ro_w2, ro_b2, w3p, b3p)
    delay = _sc_delay(y, cap_pad, scl_pad)
    od = _sc_out_gather(delay, a_pad)
    return od[:NP_].reshape(NP_, 1)


# speedup vs baseline: 2.4336x; 1.0507x over previous
import functools

import jax
import jax.numpy as jnp
from jax import lax
from jax.experimental import pallas as pl
from jax.experimental.pallas import tpu as pltpu
from jax.experimental.pallas import tpu_sc as plsc

NP_ = 100000
NL = 20000
D = 64
NW = 32
PT = 3136
NPP = NW * PT
NV = 128
RNG = 160
LPAD = NV * RNG
PCAP = 102400
CHUNK = 512
RCH = 128
DW = 128
NEG = -3.402823e38
POS = 3.402823e38

_mesh = plsc.VectorSubcoreMesh(core_axis_name="c", subcore_axis_name="s")


def _wid():
    return lax.axis_index("s") * 2 + lax.axis_index("c")


def _iota16():
    return lax.broadcasted_iota(jnp.int32, (16,), 0)


def _ds8(off, n):
    return pl.ds(pl.multiple_of(off, 8), n)


@functools.partial(
    pl.kernel,
    out_type=(
        jax.ShapeDtypeStruct((NV * PCAP,), jnp.int32),
        jax.ShapeDtypeStruct((NV * 16,), jnp.int32),
        jax.ShapeDtypeStruct((LPAD,), jnp.float32),
    ),
    mesh=_mesh,
    compiler_params=pltpu.CompilerParams(needs_layout_passes=False),
    scratch_types=[
        pltpu.VMEM((2000,), jnp.int32),
        pltpu.VMEM((4096,), jnp.int32),
        pltpu.VMEM((RNG,), jnp.int32),
        pltpu.VMEM((RNG,), jnp.float32),
        pltpu.VMEM((16,), jnp.int32),
    ],
)
def _sc_plan(sl_hbm, plan_hbm, nv_hbm, cnt_hbm, slv, pbuf, cacc, cf, nbuf):
    wid = _wid()
    iota = _iota16()
    ones = jnp.ones((16,), jnp.int32)
    for voff in (0, 32, 64, 96):
        v = wid + voff
        lo = v * RNG

        def zero_cnt(j, _):
            cacc[pl.ds(j * 16, 16)] = jnp.zeros((16,), jnp.int32)
            return 0

        lax.fori_loop(0, RNG // 16, zero_cnt, 0)

        def chunk(c, carry):
            ptr0, hb0 = carry
            pltpu.sync_copy(sl_hbm.at[_ds8(c * 2000, 2000)], slv)

            def step(i, ptr):
                sl = slv[pl.ds(i * 16, 16)]
                m = (sl >= lo) & (sl < lo + RNG)
                ll = jnp.clip(sl - lo, 0, RNG - 1)
                pid = c * 2000 + i * 16 + iota
                packed = (pid << 9) | ll
                mi32 = m.astype(jnp.int32)
                pos = ptr + plsc.cumsum(mi32) - 1
                plsc.store_scatter(pbuf, [pos], packed, mask=m)
                plsc.addupdate_scatter(cacc, [ll], ones, mask=m)
                return ptr + jnp.sum(mi32)

            ptr = lax.fori_loop(0, 125, step, ptr0)

            def flush(args):
                p, hb = args
                pltpu.sync_copy(pbuf.at[pl.ds(0, 2048)],
                                plan_hbm.at[_ds8(v * PCAP + hb, 2048)])

                def mv(i, _):
                    pbuf[pl.ds(i * 16, 16)] = pbuf[pl.ds(2048 + i * 16, 16)]
                    return 0

                lax.fori_loop(0, 128, mv, 0)
                return (p - 2048, hb + 2048)

            return lax.cond(ptr >= 2048, flush, lambda a: a, (ptr, hb0))

        ptr, hbase = lax.fori_loop(0, 50, chunk,
                                   (jnp.int32(0), jnp.int32(0)))

        def ztail(i, _):
            pbuf[pl.ds(ptr + i * 16, 16)] = jnp.full((16,), RNG, jnp.int32)
            return 0

        lax.fori_loop(0, 32, ztail, 0)
        pltpu.sync_copy(pbuf.at[pl.ds(0, 2048)],
                        plan_hbm.at[_ds8(v * PCAP + hbase, 2048)])
        pltpu.sync_copy(pbuf.at[pl.ds(2048, 512)],
                        plan_hbm.at[_ds8(v * PCAP + hbase + 2048, 512)])
        nbuf[...] = jnp.where(iota == 0, ptr + hbase, 0)
        pltpu.sync_copy(nbuf, nv_hbm.at[_ds8(v * 16, 16)])

        def cvt(j, _):
            cf[pl.ds(j * 16, 16)] = cacc[pl.ds(j * 16, 16)].astype(jnp.float32)
            return 0

        lax.fori_loop(0, RNG // 16, cvt, 0)
        pltpu.sync_copy(cf, cnt_hbm.at[_ds8(lo, RNG)])


@functools.partial(
    pl.kernel,
    out_type=jax.ShapeDtypeStruct((NPP, DW), jnp.float32),
    mesh=_mesh,
    compiler_params=pltpu.CompilerParams(needs_layout_passes=False),
    scratch_types=[
        pltpu.VMEM((CHUNK,), jnp.int32),
        pltpu.VMEM((CHUNK, DW), jnp.float32),
        pltpu.VMEM((64,), jnp.int32),
        pltpu.VMEM((64, DW), jnp.float32),
        pltpu.SemaphoreType.DMA,
    ],
)
def _sc_gather(idx_hbm, tbl_hbm, out_hbm, idxv, rows, idxv2, rows2, sem):
    base = _wid() * PT
    for off in (0, 512, 1024, 1536, 2048, 2560):
        pltpu.sync_copy(idx_hbm.at[_ds8(base + off, CHUNK)], idxv)
        pltpu.async_copy(tbl_hbm.at[idxv], rows, sem).wait()
        pltpu.sync_copy(rows, out_hbm.at[_ds8(base + off, CHUNK)])
    pltpu.sync_copy(idx_hbm.at[_ds8(base + 3072, 64)], idxv2)
    pltpu.async_copy(tbl_hbm.at[idxv2], rows2, sem).wait()
    pltpu.sync_copy(rows2, out_hbm.at[_ds8(base + 3072, 64)])


@functools.partial(
    pl.kernel,
    out_type=(
        jax.ShapeDtypeStruct((LPAD, D), jnp.float32),
        jax.ShapeDtypeStruct((LPAD, D), jnp.float32),
        jax.ShapeDtypeStruct((LPAD, D), jnp.float32),
    ),
    mesh=_mesh,
    compiler_params=pltpu.CompilerParams(needs_layout_passes=False),
    scratch_types=[
        pltpu.VMEM((RCH,), jnp.int32),
        pltpu.VMEM((RCH,), jnp.int32),
        pltpu.VMEM((RCH, DW), jnp.float32),
        pltpu.VMEM((RNG + 1, D), jnp.float32),
        pltpu.VMEM((RNG + 1, D), jnp.float32),
        pltpu.VMEM((RNG + 1, D), jnp.float32),
        pltpu.VMEM((16,), jnp.int32),
        pltpu.SemaphoreType.DMA,
    ],
)
def _sc_reduce(h_hbm, plan_hbm, nv_hbm, sum_hbm, max_hbm, min_hbm,
               pk, pidx, rows, asum, amax, amin, nbuf, sem):
    wid = _wid()
    iota = _iota16()
    cols = [iota + r * 16 for r in range(4)]
    zf = jnp.zeros((16,), jnp.float32)
    negf = jnp.full((16,), NEG, jnp.float32)
    posf = jnp.full((16,), POS, jnp.float32)
    for voff in (0, 32, 64, 96):
        v = wid + voff
        lo = v * RNG
        pltpu.sync_copy(nv_hbm.at[_ds8(v * 16, 16)], nbuf)
        n = nbuf[...][0]

        def zrow(l, _):
            rowl = jnp.full((16,), l, jnp.int32)
            for r in range(4):
                plsc.store_scatter(asum, [rowl, cols[r]], zf)
                plsc.store_scatter(amax, [rowl, cols[r]], negf)
                plsc.store_scatter(amin, [rowl, cols[r]], posf)
            return 0

        lax.fori_loop(0, RNG + 1, zrow, 0)

        def chunk(c, _):
            pltpu.sync_copy(plan_hbm.at[_ds8(v * PCAP + c * RCH, RCH)], pk)

            def unp(i, _):
                pidx[pl.ds(i * 16, 16)] = pk[pl.ds(i * 16, 16)] >> 9
                return 0

            lax.fori_loop(0, RCH // 16, unp, 0)
            pltpu.async_copy(h_hbm.at[pidx], rows, sem).wait()

            def rmw(i, _):
                pkv = pk[pl.ds(i * 16, 16)]
                lv = pkv & 511
                for lane in range(16):
                    l = lv[lane]
                    rowj = jnp.full((16,), i * 16 + lane, jnp.int32)
                    rowl = jnp.full((16,), l, jnp.int32)
                    hv = [plsc.load_gather(rows, [rowj, cols[r]])
                          for r in range(4)]
                    mv = [plsc.load_gather(amax, [rowl, cols[r]])
                          for r in range(4)]
                    nv_ = [plsc.load_gather(amin, [rowl, cols[r]])
                           for r in range(4)]
                    for r in range(4):
                        plsc.addupdate_scatter(asum, [rowl, cols[r]], hv[r])
                    for r in range(4):
                        plsc.store_scatter(amax, [rowl, cols[r]],
                                           jnp.maximum(mv[r], hv[r]))
                    for r in range(4):
                        plsc.store_scatter(amin, [rowl, cols[r]],
                                           jnp.minimum(nv_[r], hv[r]))
                return 0

            lax.fori_loop(0, RCH // 16, rmw, 0)
            return 0

        lax.fori_loop(0, (n + RCH - 1) // RCH, chunk, 0)
        pltpu.sync_copy(asum.at[pl.ds(0, RNG)], sum_hbm.at[_ds8(lo, RNG)])
        pltpu.sync_copy(amax.at[pl.ds(0, RNG)], max_hbm.at[_ds8(lo, RNG)])
        pltpu.sync_copy(amin.at[pl.ds(0, RNG)], min_hbm.at[_ds8(lo, RNG)])


@functools.partial(
    pl.kernel,
    out_type=jax.ShapeDtypeStruct((LPAD,), jnp.float32),
    mesh=_mesh,
    compiler_params=pltpu.CompilerParams(needs_layout_passes=False),
    scratch_types=[
        pltpu.VMEM((LPAD // NW, 8), jnp.float32),
        pltpu.VMEM((LPAD // NW,), jnp.float32),
        pltpu.VMEM((LPAD // NW,), jnp.float32),
        pltpu.VMEM((LPAD // NW,), jnp.float32),
    ],
)
def _sc_delay(y_hbm, cap_hbm, scl_hbm, out_hbm, yv, cv, sv, dbuf):
    per = LPAD // NW
    lo = _wid() * per
    iota = _iota16()
    zc = jnp.zeros((16,), jnp.int32)
    pltpu.sync_copy(y_hbm.at[_ds8(lo, per)], yv)
    pltpu.sync_copy(cap_hbm.at[_ds8(lo, per)], cv)
    pltpu.sync_copy(scl_hbm.at[_ds8(lo, per)], sv)

    def step(j, _):
        rows = j * 16 + iota
        y0 = plsc.load_gather(yv, [rows, zc])
        occ = 1.0 / (1.0 + jnp.exp(-y0))
        c = cv[pl.ds(j * 16, 16)]
        s = sv[pl.ds(j * 16, 16)]
        rc = (c * 24631.01 + 21166.35) * (s * 5.77 + 10.5)
        qd = occ * 32000.0 / rc
        qd = jnp.where(qd != qd, 0.0, qd)
        td = 1000.0 / rc
        td = jnp.where(jnp.abs(td) == jnp.inf, 0.0, td)
        dbuf[pl.ds(j * 16, 16)] = qd + td
        return 0

    lax.fori_loop(0, per // 16, step, 0)
    pltpu.sync_copy(dbuf, out_hbm.at[_ds8(lo, per)])


@functools.partial(
    pl.kernel,
    out_type=jax.ShapeDtypeStruct((NPP,), jnp.float32),
    mesh=_mesh,
    compiler_params=pltpu.CompilerParams(needs_layout_passes=False),
    scratch_types=[
        pltpu.VMEM((LPAD,), jnp.float32),
        pltpu.VMEM((CHUNK,), jnp.int32),
        pltpu.VMEM((CHUNK,), jnp.float32),
        pltpu.VMEM((64,), jnp.int32),
        pltpu.VMEM((64,), jnp.float32),
    ],
)
def _sc_out_gather(dl_hbm, idx_hbm, out_hbm, tbl, idxv, obuf, idxv2, obuf2):
    base = _wid() * PT
    pltpu.sync_copy(dl_hbm, tbl)
    for off in (0, 512, 1024, 1536, 2048, 2560):
        pltpu.sync_copy(idx_hbm.at[_ds8(base + off, CHUNK)], idxv)

        def step(i, _):
            av = idxv[pl.ds(i * 16, 16)]
            obuf[pl.ds(i * 16, 16)] = plsc.load_gather(tbl, [av])
            return 0

        lax.fori_loop(0, CHUNK // 16, step, 0)
        pltpu.sync_copy(obuf, out_hbm.at[_ds8(base + off, CHUNK)])
    pltpu.sync_copy(idx_hbm.at[_ds8(base + 3072, 64)], idxv2)

    def step2(i, _):
        av = idxv2[pl.ds(i * 16, 16)]
        obuf2[pl.ds(i * 16, 16)] = plsc.load_gather(tbl, [av])
        return 0

    lax.fori_loop(0, 4, step2, 0)
    pltpu.sync_copy(obuf2, out_hbm.at[_ds8(base + 3072, 64)])


def _full(shape):
    return pl.BlockSpec(shape, lambda i: tuple(0 for _ in shape))


def _tc_link_embed_body(cs, w1, b1, w2, b2, out):
    t = jnp.maximum(jnp.dot(cs[...], w1[...],
                            preferred_element_type=jnp.float32) + b1[...], 0.0)
    h = jnp.maximum(jnp.dot(t, w2[...],
                            preferred_element_type=jnp.float32) + b2[...], 0.0)
    out[...] = jnp.concatenate(
        [h, jnp.zeros((h.shape[0], DW - D), jnp.float32)], axis=1)


def _tc_link_embed(cs, w1, b1, w2, b2):
    blk = 2000
    return pl.pallas_call(
        _tc_link_embed_body,
        grid=(NL // blk,),
        in_specs=[
            pl.BlockSpec((blk, 2), lambda i: (i, 0)),
            _full(w1.shape), _full(b1.shape), _full(w2.shape), _full(b2.shape),
        ],
        out_specs=pl.BlockSpec((blk, DW), lambda i: (i, 0)),
        out_shape=jax.ShapeDtypeStruct((NL, DW), jnp.float32),
    )(cs, w1, b1, w2, b2)


def _gru_math(mx, mh, h):
    xz, xr, xh = mx[:, :D], mx[:, D:2 * D], mx[:, 2 * D:]
    hz, hr, hh = mh[:, :D], mh[:, D:2 * D], mh[:, 2 * D:]
    z = jax.nn.sigmoid(xz + hz)
    r = jax.nn.sigmoid(xr + hr)
    cand = jnp.tanh(xh + r * hh)
    return z * h + (1.0 - z) * cand


def _tc_pgru_body(tp, g, pw1, pb1, pw2, pb2, wi, wr, bi, br, out):
    t = jnp.maximum(jnp.dot(tp[...], pw1[...],
                            preferred_element_type=jnp.float32) + pb1[...], 0.0)
    h0 = jnp.maximum(jnp.dot(t, pw2[...],
                             preferred_element_type=jnp.float32) + pb2[...], 0.0)
    gv = g[...][:, :D]
    mx = jnp.dot(gv, wi[...], preferred_element_type=jnp.float32) + bi[...]
    mh = jnp.dot(h0, wr[...], preferred_element_type=jnp.float32) + br[...]
    h = _gru_math(mx, mh, h0)
    nz = jnp.sum((gv != 0.0).astype(jnp.float32), axis=1, keepdims=True)
    h = jnp.where(nz > 0.0, h, h0)
    out[...] = jnp.concatenate(
        [h, jnp.zeros((h.shape[0], DW - D), jnp.float32)], axis=1)


def _tc_pgru(tp, g, pw1, pb1, pw2, pb2, wi, wr, bi, br):
    blk = 2000
    return pl.pallas_call(
        _tc_pgru_body,
        grid=(NP_ // blk,),
        in_specs=[
            pl.BlockSpec((blk, 2), lambda i: (i, 0)),
            pl.BlockSpec((blk, DW), lambda i: (i, 0)),
            _full(pw1.shape), _full(pb1.shape),
            _full(pw2.shape), _full(pb2.shape),
            _full(wi.shape), _full(wr.shape),
            _full(bi.shape), _full(br.shape),
        ],
        out_specs=pl.BlockSpec((blk, DW), lambda i: (i, 0)),
        out_shape=jax.ShapeDtypeStruct((NP_, DW), jnp.float32),
    )(tp, g, pw1, pb1, pw2, pb2, wi, wr, bi, br)


def _tc_link_body(s, mx_, mn_, cnt, ls, w1, b1, w2, b2, w3, b3,
                  wi, wr, bi, br, out):
    sv = s[...]
    cntv = cnt[...]
    has = cntv > 0.0
    mean = jnp.where(has, sv / jnp.maximum(cntv, 1.0), 0.0)
    mxv = jnp.where(has, mx_[...], 0.0)
    mnv = jnp.where(has, mn_[...], 0.0)
    mi = jnp.concatenate([sv, mean, mxv, mnv], axis=1)
    pa = jnp.maximum(jnp.dot(mi, w1[...],
                             preferred_element_type=jnp.float32) + b1[...], 0.0)
    pa = jnp.maximum(jnp.dot(pa, w2[...],
                             preferred_element_type=jnp.float32) + b2[...], 0.0)
    pa = jnp.maximum(jnp.dot(pa, w3[...],
                             preferred_element_type=jnp.float32) + b3[...], 0.0)
    h = ls[...][:, :D]
    mxg = jnp.dot(pa, wi[...], preferred_element_type=jnp.float32) + bi[...]
    mhg = jnp.dot(h, wr[...], preferred_element_type=jnp.float32) + br[...]
    hn = _gru_math(mxg, mhg, h)
    out[...] = jnp.concatenate(
        [hn, jnp.zeros((hn.shape[0], DW - D), jnp.float32)], axis=1)


def _tc_link(s, mx_, mn_, cnt2d, ls, w1, b1, w2, b2, w3, b3, wi, wr, bi, br):
    blk = 2000
    return pl.pallas_call(
        _tc_link_body,
        grid=(NL // blk,),
        in_specs=[
            pl.BlockSpec((blk, D), lambda i: (i, 0)),
            pl.BlockSpec((blk, D), lambda i: (i, 0)),
            pl.BlockSpec((blk, D), lambda i: (i, 0)),
            pl.BlockSpec((blk, 1), lambda i: (i, 0)),
            pl.BlockSpec((blk, DW), lambda i: (i, 0)),
            _full(w1.shape), _full(b1.shape), _full(w2.shape), _full(b2.shape),
            _full(w3.shape), _full(b3.shape),
            _full(wi.shape), _full(wr.shape), _full(bi.shape), _full(br.shape),
        ],
        out_specs=pl.BlockSpec((blk, DW), lambda i: (i, 0)),
        out_shape=jax.ShapeDtypeStruct((NL, DW), jnp.float32),
    )(s, mx_, mn_, cnt2d, ls, w1, b1, w2, b2, w3, b3, wi, wr, bi, br)


def _tc_readout_body(ls, w1, b1, w2, b2, w3p, b3p, out):
    o = jnp.maximum(jnp.dot(ls[...][:, :D], w1[...],
                            preferred_element_type=jnp.float32) + b1[...], 0.0)
    o = jnp.maximum(jnp.dot(o, w2[...],
                            preferred_element_type=jnp.float32) + b2[...], 0.0)
    out[...] = jnp.dot(o, w3p[...],
                       preferred_element_type=jnp.float32) + b3p[...]


def _tc_readout(ls, w1, b1, w2, b2, w3p, b3p):
    blk = 2000
    return pl.pallas_call(
        _tc_readout_body,
        grid=(NL // blk,),
        in_specs=[
            pl.BlockSpec((blk, DW), lambda i: (i, 0)),
            _full(w1.shape), _full(b1.shape), _full(w2.shape), _full(b2.shape),
            _full(w3p.shape), _full(b3p.shape),
        ],
        out_specs=pl.BlockSpec((blk, 8), lambda i: (i, 0)),
        out_shape=jax.ShapeDtypeStruct((LPAD, 8), jnp.float32),
    )(ls, w1, b1, w2, b2, w3p, b3p)


def kernel(traffic, packets, capacity, scale, le_w1, le_b1, le_w2, le_b2,
           pe_w1, pe_b1, pe_w2, pe_b2, am_w1, am_b1, am_w2, am_b2, am_w3,
           am_b3, ro_w1, ro_b1, ro_w2, ro_b2, ro_w3, ro_b3, lg_wi, lg_wr,
           lg_bi, lg_br, pg_wi, pg_wr, pg_bi, pg_br, link_to_path, path_ids,
           sequence_path, sequence_links, n_links, n_paths):
    cs = jnp.stack([capacity, scale], axis=1)
    tp = jnp.stack([traffic, packets], axis=1)
    a_pad = jnp.pad(link_to_path, (0, NPP - NP_))
    cap_pad = jnp.pad(capacity, (0, LPAD - NL))
    scl_pad = jnp.pad(scale, (0, LPAD - NL))
    w3p = jnp.pad(ro_w3, ((0, 0), (0, 7)))
    b3p = jnp.pad(ro_b3, (0, 7))

    ls = _tc_link_embed(cs, le_w1, le_b1, le_w2, le_b2)
    plan, nva, counts = _sc_plan(sequence_links)
    cnt2d = counts[:NL].reshape(NL, 1)

    for _ in range(8):
        g = _sc_gather(a_pad, ls)
        h = _tc_pgru(tp, g, pe_w1, pe_b1, pe_w2, pe_b2,
                     pg_wi, pg_wr, pg_bi, pg_br)
        s, mx_, mn_ = _sc_reduce(h, plan, nva)
        ls = _tc_link(s, mx_, mn_, cnt2d, ls,
                      am_w1, am_b1, am_w2, am_b2, am_w3, am_b3,
                      lg_wi, lg_wr, lg_bi, lg_br)

    y = _tc_readout(ls, ro_w1, ro_b1, ro_w2, ro_b2, w3p, b3p)
    delay = _sc_delay(y, cap_pad, scl_pad)
    od = _sc_out_gather(delay, a_pad)
    return od[:NP_].reshape(NP_, 1)


# speedup vs baseline: 2.4710x; 1.0154x over previous
import functools

import jax
import jax.numpy as jnp
from jax import lax
from jax.experimental import pallas as pl
from jax.experimental.pallas import tpu as pltpu
from jax.experimental.pallas import tpu_sc as plsc

NP_ = 100000
NL = 20000
D = 64
NW = 32
PT = 3136
NPP = NW * PT
NV = 128
RNG = 160
LPAD = NV * RNG
PCAP = 102400
CHUNK = 512
RCH = 128
DW = 128
NEG = -3.402823e38
POS = 3.402823e38

_mesh = plsc.VectorSubcoreMesh(core_axis_name="c", subcore_axis_name="s")


def _wid():
    return lax.axis_index("s") * 2 + lax.axis_index("c")


def _iota16():
    return lax.broadcasted_iota(jnp.int32, (16,), 0)


def _ds8(off, n):
    return pl.ds(pl.multiple_of(off, 8), n)


@functools.partial(
    pl.kernel,
    out_type=(
        jax.ShapeDtypeStruct((NV * PCAP,), jnp.int32),
        jax.ShapeDtypeStruct((NV * PCAP,), jnp.int32),
        jax.ShapeDtypeStruct((NV * 16,), jnp.int32),
        jax.ShapeDtypeStruct((LPAD,), jnp.float32),
    ),
    mesh=_mesh,
    compiler_params=pltpu.CompilerParams(needs_layout_passes=False),
    scratch_types=[
        pltpu.VMEM((2000,), jnp.int32),
        pltpu.VMEM((4096,), jnp.int32),
        pltpu.VMEM((4096,), jnp.int32),
        pltpu.VMEM((RNG,), jnp.int32),
        pltpu.VMEM((RNG,), jnp.float32),
        pltpu.VMEM((16,), jnp.int32),
    ],
)
def _sc_plan(sl_hbm, pid_hbm, ll_hbm, nv_hbm, cnt_hbm,
             slv, pbuf, lbuf, cacc, cf, nbuf):
    wid = _wid()
    iota = _iota16()
    ones = jnp.ones((16,), jnp.int32)
    for voff in (0, 32, 64, 96):
        v = wid + voff
        lo = v * RNG

        def zero_cnt(j, _):
            cacc[pl.ds(j * 16, 16)] = jnp.zeros((16,), jnp.int32)
            return 0

        lax.fori_loop(0, RNG // 16, zero_cnt, 0)

        def chunk(c, carry):
            ptr0, hb0 = carry
            pltpu.sync_copy(sl_hbm.at[_ds8(c * 2000, 2000)], slv)

            def step(i, ptr):
                sl = slv[pl.ds(i * 16, 16)]
                m = (sl >= lo) & (sl < lo + RNG)
                ll = jnp.clip(sl - lo, 0, RNG - 1)
                pid = c * 2000 + i * 16 + iota
                mi32 = m.astype(jnp.int32)
                pos = ptr + plsc.cumsum(mi32) - 1
                plsc.store_scatter(pbuf, [pos], pid, mask=m)
                plsc.store_scatter(lbuf, [pos], ll, mask=m)
                plsc.addupdate_scatter(cacc, [ll], ones, mask=m)
                return ptr + jnp.sum(mi32)

            ptr = lax.fori_loop(0, 125, step, ptr0)

            def flush(args):
                p, hb = args
                pltpu.sync_copy(pbuf.at[pl.ds(0, 2048)],
                                pid_hbm.at[_ds8(v * PCAP + hb, 2048)])
                pltpu.sync_copy(lbuf.at[pl.ds(0, 2048)],
                                ll_hbm.at[_ds8(v * PCAP + hb, 2048)])

                def mv(i, _):
                    pbuf[pl.ds(i * 16, 16)] = pbuf[pl.ds(2048 + i * 16, 16)]
                    lbuf[pl.ds(i * 16, 16)] = lbuf[pl.ds(2048 + i * 16, 16)]
                    return 0

                lax.fori_loop(0, 128, mv, 0)
                return (p - 2048, hb + 2048)

            return lax.cond(ptr >= 2048, flush, lambda a: a, (ptr, hb0))

        ptr, hbase = lax.fori_loop(0, 50, chunk,
                                   (jnp.int32(0), jnp.int32(0)))

        def ztail(i, _):
            pbuf[pl.ds(ptr + i * 16, 16)] = jnp.zeros((16,), jnp.int32)
            lbuf[pl.ds(ptr + i * 16, 16)] = jnp.full((16,), RNG, jnp.int32)
            return 0

        lax.fori_loop(0, 32, ztail, 0)
        pltpu.sync_copy(pbuf.at[pl.ds(0, 2048)],
                        pid_hbm.at[_ds8(v * PCAP + hbase, 2048)])
        pltpu.sync_copy(pbuf.at[pl.ds(2048, 512)],
                        pid_hbm.at[_ds8(v * PCAP + hbase + 2048, 512)])
        pltpu.sync_copy(lbuf.at[pl.ds(0, 2048)],
                        ll_hbm.at[_ds8(v * PCAP + hbase, 2048)])
        pltpu.sync_copy(lbuf.at[pl.ds(2048, 512)],
                        ll_hbm.at[_ds8(v * PCAP + hbase + 2048, 512)])
        nbuf[...] = jnp.where(iota == 0, ptr + hbase, 0)
        pltpu.sync_copy(nbuf, nv_hbm.at[_ds8(v * 16, 16)])

        def cvt(j, _):
            cf[pl.ds(j * 16, 16)] = cacc[pl.ds(j * 16, 16)].astype(jnp.float32)
            return 0

        lax.fori_loop(0, RNG // 16, cvt, 0)
        pltpu.sync_copy(cf, cnt_hbm.at[_ds8(lo, RNG)])


GC = 224


@functools.partial(
    pl.kernel,
    out_type=jax.ShapeDtypeStruct((NPP, DW), jnp.float32),
    mesh=_mesh,
    compiler_params=pltpu.CompilerParams(needs_layout_passes=False),
    scratch_types=[
        pltpu.VMEM((PT,), jnp.int32),
        pltpu.VMEM((GC, DW), jnp.float32),
        pltpu.VMEM((GC, DW), jnp.float32),
        pltpu.SemaphoreType.DMA,
        pltpu.SemaphoreType.DMA,
        pltpu.SemaphoreType.DMA,
        pltpu.SemaphoreType.DMA,
    ],
)
def _sc_gather(idx_hbm, tbl_hbm, out_hbm, idxall, rowsA, rowsB,
               sga, sgb, soa, sob):
    base = _wid() * PT
    pltpu.sync_copy(idx_hbm.at[_ds8(base, PT)], idxall)
    bufs = [(rowsA, sga, soa), (rowsB, sgb, sob)]
    nchunks = PT // GC
    gd = {0: pltpu.async_copy(tbl_hbm.at[idxall.at[pl.ds(0, GC)]],
                              rowsA, sga)}
    od = {}
    for k in range(nchunks):
        rows, sg, so = bufs[k % 2]
        if k + 1 < nchunks:
            nrows, nsg, _unused = bufs[(k + 1) % 2]
            if k >= 1:
                od[k - 1].wait()
            gd[k + 1] = pltpu.async_copy(
                tbl_hbm.at[idxall.at[pl.ds((k + 1) * GC, GC)]], nrows, nsg)
        gd[k].wait()
        od[k] = pltpu.async_copy(rows, out_hbm.at[_ds8(base + k * GC, GC)],
                                 so)
    od[nchunks - 2].wait()
    od[nchunks - 1].wait()


@functools.partial(
    pl.kernel,
    out_type=(
        jax.ShapeDtypeStruct((LPAD, D), jnp.float32),
        jax.ShapeDtypeStruct((LPAD, D), jnp.float32),
        jax.ShapeDtypeStruct((LPAD, D), jnp.float32),
    ),
    mesh=_mesh,
    compiler_params=pltpu.CompilerParams(needs_layout_passes=False),
    scratch_types=[
        pltpu.VMEM((2048,), jnp.int32),
        pltpu.VMEM((2048,), jnp.int32),
        pltpu.VMEM((RCH, DW), jnp.float32),
        pltpu.VMEM((RCH, DW), jnp.float32),
        pltpu.VMEM((RCH,), jnp.int32),
        pltpu.VMEM((RCH,), jnp.int32),
        pltpu.VMEM((RNG + 1, D), jnp.float32),
        pltpu.VMEM((RNG + 1, D), jnp.float32),
        pltpu.VMEM((RNG + 1, D), jnp.float32),
        pltpu.VMEM((16,), jnp.int32),
        pltpu.SemaphoreType.DMA,
        pltpu.SemaphoreType.DMA,
    ],
)
def _sc_reduce(h_hbm, pid_hbm, ll_hbm, nv_hbm, sum_hbm, max_hbm, min_hbm,
               pkb, llb, rowsA, rowsB, pidxA, pidxB,
               asum, amax, amin, nbuf, semA, semB):
    wid = _wid()
    iota = _iota16()
    cols = [iota + r * 16 for r in range(4)]
    zf = jnp.zeros((16,), jnp.float32)
    negf = jnp.full((16,), NEG, jnp.float32)
    posf = jnp.full((16,), POS, jnp.float32)

    def gstart(s, pidx, rows, sem):
        def cp(i, _):
            pidx[pl.ds(i * 16, 16)] = pkb[pl.ds(s * RCH + i * 16, 16)]
            return 0

        lax.fori_loop(0, RCH // 16, cp, 0)
        return pltpu.async_copy(h_hbm.at[pidx], rows, sem)

    def gwait(pidx, rows, sem):
        pltpu.make_async_copy(h_hbm.at[pidx], rows, sem).wait()

    def wave(w, _):
        v = wid + 32 * w
        lo = v * RNG
        pltpu.sync_copy(nv_hbm.at[_ds8(v * 16, 16)], nbuf)
        n = nbuf[...][0]

        def zrow(l, _):
            rowl = jnp.full((16,), l, jnp.int32)
            for r in range(4):
                plsc.store_scatter(asum, [rowl, cols[r]], zf)
                plsc.store_scatter(amax, [rowl, cols[r]], negf)
                plsc.store_scatter(amin, [rowl, cols[r]], posf)
            return 0

        lax.fori_loop(0, RNG + 1, zrow, 0)
        nc = (n + RCH - 1) // RCH

        def rmw(s, rows):
            def grp(i, _):
                lv = llb[pl.ds(s * RCH + i * 16, 16)]
                for lane in range(16):
                    l = lv[lane]
                    rowj = jnp.full((16,), i * 16 + lane, jnp.int32)
                    rowl = jnp.full((16,), l, jnp.int32)
                    RQ = 4
                    hv = [plsc.load_gather(rows, [rowj, cols[r]])
                          for r in range(RQ)]
                    mv = [plsc.load_gather(amax, [rowl, cols[r]])
                          for r in range(RQ)]
                    nv_ = [plsc.load_gather(amin, [rowl, cols[r]])
                           for r in range(RQ)]
                    for r in range(RQ):
                        plsc.addupdate_scatter(asum, [rowl, cols[r]], hv[r])
                    for r in range(RQ):
                        plsc.store_scatter(amax, [rowl, cols[r]],
                                           jnp.maximum(mv[r], hv[r]))
                    for r in range(RQ):
                        plsc.store_scatter(amin, [rowl, cols[r]],
                                           jnp.minimum(nv_[r], hv[r]))
                return 0

            lax.fori_loop(0, RCH // 16, grp, 0)

        def block(t, _):
            bbase = v * PCAP + t * 2048
            pltpu.sync_copy(pid_hbm.at[_ds8(bbase, 2048)], pkb)
            pltpu.sync_copy(ll_hbm.at[_ds8(bbase, 2048)], llb)
            ns = jnp.minimum(2048 // RCH, nc - t * (2048 // RCH))
            gstart(0, pidxA, rowsA, semA)

            def pair(p, _):
                s0 = p * 2
                s1 = s0 + 1

                @pl.when(s1 < ns)
                def _():
                    gstart(s1, pidxB, rowsB, semB)

                gwait(pidxA, rowsA, semA)
                rmw(s0, rowsA)

                @pl.when(s0 + 2 < ns)
                def _():
                    gstart(s0 + 2, pidxA, rowsA, semA)

                @pl.when(s1 < ns)
                def _():
                    gwait(pidxB, rowsB, semB)
                    rmw(s1, rowsB)

                return 0

            lax.fori_loop(0, (ns + 1) // 2, pair, 0)
            return 0

        lax.fori_loop(0, (n + 2047) // 2048, block, 0)
        pltpu.sync_copy(asum.at[pl.ds(0, RNG)], sum_hbm.at[_ds8(lo, RNG)])
        pltpu.sync_copy(amax.at[pl.ds(0, RNG)], max_hbm.at[_ds8(lo, RNG)])
        pltpu.sync_copy(amin.at[pl.ds(0, RNG)], min_hbm.at[_ds8(lo, RNG)])
        return 0

    lax.fori_loop(0, 4, wave, 0)


@functools.partial(
    pl.kernel,
    out_type=jax.ShapeDtypeStruct((LPAD,), jnp.float32),
    mesh=_mesh,
    compiler_params=pltpu.CompilerParams(needs_layout_passes=False),
    scratch_types=[
        pltpu.VMEM((LPAD // NW, 8), jnp.float32),
        pltpu.VMEM((LPAD // NW,), jnp.float32),
        pltpu.VMEM((LPAD // NW,), jnp.float32),
        pltpu.VMEM((LPAD // NW,), jnp.float32),
    ],
)
def _sc_delay(y_hbm, cap_hbm, scl_hbm, out_hbm, yv, cv, sv, dbuf):
    per = LPAD // NW
    lo = _wid() * per
    iota = _iota16()
    zc = jnp.zeros((16,), jnp.int32)
    pltpu.sync_copy(y_hbm.at[_ds8(lo, per)], yv)
    pltpu.sync_copy(cap_hbm.at[_ds8(lo, per)], cv)
    pltpu.sync_copy(scl_hbm.at[_ds8(lo, per)], sv)

    def step(j, _):
        rows = j * 16 + iota
        y0 = plsc.load_gather(yv, [rows, zc])
        occ = 1.0 / (1.0 + jnp.exp(-y0))
        c = cv[pl.ds(j * 16, 16)]
        s = sv[pl.ds(j * 16, 16)]
        rc = (c * 24631.01 + 21166.35) * (s * 5.77 + 10.5)
        qd = occ * 32000.0 / rc
        qd = jnp.where(qd != qd, 0.0, qd)
        td = 1000.0 / rc
        td = jnp.where(jnp.abs(td) == jnp.inf, 0.0, td)
        dbuf[pl.ds(j * 16, 16)] = qd + td
        return 0

    lax.fori_loop(0, per // 16, step, 0)
    pltpu.sync_copy(dbuf, out_hbm.at[_ds8(lo, per)])


@functools.partial(
    pl.kernel,
    out_type=jax.ShapeDtypeStruct((NPP,), jnp.float32),
    mesh=_mesh,
    compiler_params=pltpu.CompilerParams(needs_layout_passes=False),
    scratch_types=[
        pltpu.VMEM((LPAD,), jnp.float32),
        pltpu.VMEM((CHUNK,), jnp.int32),
        pltpu.VMEM((CHUNK,), jnp.float32),
        pltpu.VMEM((64,), jnp.int32),
        pltpu.VMEM((64,), jnp.float32),
    ],
)
def _sc_out_gather(dl_hbm, idx_hbm, out_hbm, tbl, idxv, obuf, idxv2, obuf2):
    base = _wid() * PT
    pltpu.sync_copy(dl_hbm, tbl)
    for off in (0, 512, 1024, 1536, 2048, 2560):
        pltpu.sync_copy(idx_hbm.at[_ds8(base + off, CHUNK)], idxv)

        def step(i, _):
            av = idxv[pl.ds(i * 16, 16)]
            obuf[pl.ds(i * 16, 16)] = plsc.load_gather(tbl, [av])
            return 0

        lax.fori_loop(0, CHUNK // 16, step, 0)
        pltpu.sync_copy(obuf, out_hbm.at[_ds8(base + off, CHUNK)])
    pltpu.sync_copy(idx_hbm.at[_ds8(base + 3072, 64)], idxv2)

    def step2(i, _):
        av = idxv2[pl.ds(i * 16, 16)]
        obuf2[pl.ds(i * 16, 16)] = plsc.load_gather(tbl, [av])
        return 0

    lax.fori_loop(0, 4, step2, 0)
    pltpu.sync_copy(obuf2, out_hbm.at[_ds8(base + 3072, 64)])


def _full(shape):
    return pl.BlockSpec(shape, lambda i: tuple(0 for _ in shape))


def _tc_link_embed_body(cs, w1, b1, w2, b2, out):
    t = jnp.maximum(jnp.dot(cs[...], w1[...],
                            preferred_element_type=jnp.float32) + b1[...], 0.0)
    h = jnp.maximum(jnp.dot(t, w2[...],
                            preferred_element_type=jnp.float32) + b2[...], 0.0)
    out[...] = jnp.concatenate(
        [h, jnp.zeros((h.shape[0], DW - D), jnp.float32)], axis=1)


def _tc_link_embed(cs, w1, b1, w2, b2):
    blk = 2000
    return pl.pallas_call(
        _tc_link_embed_body,
        grid=(NL // blk,),
        in_specs=[
            pl.BlockSpec((blk, 2), lambda i: (i, 0)),
            _full(w1.shape), _full(b1.shape), _full(w2.shape), _full(b2.shape),
        ],
        out_specs=pl.BlockSpec((blk, DW), lambda i: (i, 0)),
        out_shape=jax.ShapeDtypeStruct((NL, DW), jnp.float32),
    )(cs, w1, b1, w2, b2)


def _gru_math(mx, mh, h):
    xz, xr, xh = mx[:, :D], mx[:, D:2 * D], mx[:, 2 * D:]
    hz, hr, hh = mh[:, :D], mh[:, D:2 * D], mh[:, 2 * D:]
    z = jax.nn.sigmoid(xz + hz)
    r = jax.nn.sigmoid(xr + hr)
    cand = jnp.tanh(xh + r * hh)
    return z * h + (1.0 - z) * cand


def _tc_pgru_body(tp, g, pw1, pb1, pw2, pb2, wi, wr, bi, br, out):
    t = jnp.maximum(jnp.dot(tp[...], pw1[...],
                            preferred_element_type=jnp.float32) + pb1[...], 0.0)
    h0 = jnp.maximum(jnp.dot(t, pw2[...],
                             preferred_element_type=jnp.float32) + pb2[...], 0.0)
    gv = g[...][:, :D]
    mx = jnp.dot(gv, wi[...], preferred_element_type=jnp.float32) + bi[...]
    mh = jnp.dot(h0, wr[...], preferred_element_type=jnp.float32) + br[...]
    h = _gru_math(mx, mh, h0)
    nz = jnp.sum((gv != 0.0).astype(jnp.float32), axis=1, keepdims=True)
    h = jnp.where(nz > 0.0, h, h0)
    out[...] = jnp.concatenate(
        [h, jnp.zeros((h.shape[0], DW - D), jnp.float32)], axis=1)


def _tc_pgru(tp, g, pw1, pb1, pw2, pb2, wi, wr, bi, br):
    blk = 2000
    return pl.pallas_call(
        _tc_pgru_body,
        grid=(NP_ // blk,),
        in_specs=[
            pl.BlockSpec((blk, 2), lambda i: (i, 0)),
            pl.BlockSpec((blk, DW), lambda i: (i, 0)),
            _full(pw1.shape), _full(pb1.shape),
            _full(pw2.shape), _full(pb2.shape),
            _full(wi.shape), _full(wr.shape),
            _full(bi.shape), _full(br.shape),
        ],
        out_specs=pl.BlockSpec((blk, DW), lambda i: (i, 0)),
        out_shape=jax.ShapeDtypeStruct((NP_, DW), jnp.float32),
    )(tp, g, pw1, pb1, pw2, pb2, wi, wr, bi, br)


def _tc_link_body(s, mx_, mn_, cnt, ls, w1, b1, w2, b2, w3, b3,
                  wi, wr, bi, br, out):
    sv = s[...]
    cntv = cnt[...]
    has = cntv > 0.0
    mean = jnp.where(has, sv / jnp.maximum(cntv, 1.0), 0.0)
    mxv = jnp.where(has, mx_[...], 0.0)
    mnv = jnp.where(has, mn_[...], 0.0)
    mi = jnp.concatenate([sv, mean, mxv, mnv], axis=1)
    pa = jnp.maximum(jnp.dot(mi, w1[...],
                             preferred_element_type=jnp.float32) + b1[...], 0.0)
    pa = jnp.maximum(jnp.dot(pa, w2[...],
                             preferred_element_type=jnp.float32) + b2[...], 0.0)
    pa = jnp.maximum(jnp.dot(pa, w3[...],
                             preferred_element_type=jnp.float32) + b3[...], 0.0)
    h = ls[...][:, :D]
    mxg = jnp.dot(pa, wi[...], preferred_element_type=jnp.float32) + bi[...]
    mhg = jnp.dot(h, wr[...], preferred_element_type=jnp.float32) + br[...]
    hn = _gru_math(mxg, mhg, h)
    out[...] = jnp.concatenate(
        [hn, jnp.zeros((hn.shape[0], DW - D), jnp.float32)], axis=1)


def _tc_link(s, mx_, mn_, cnt2d, ls, w1, b1, w2, b2, w3, b3, wi, wr, bi, br):
    blk = 2000
    return pl.pallas_call(
        _tc_link_body,
        grid=(NL // blk,),
        in_specs=[
            pl.BlockSpec((blk, D), lambda i: (i, 0)),
            pl.BlockSpec((blk, D), lambda i: (i, 0)),
            pl.BlockSpec((blk, D), lambda i: (i, 0)),
            pl.BlockSpec((blk, 1), lambda i: (i, 0)),
            pl.BlockSpec((blk, DW), lambda i: (i, 0)),
            _full(w1.shape), _full(b1.shape), _full(w2.shape), _full(b2.shape),
            _full(w3.shape), _full(b3.shape),
            _full(wi.shape), _full(wr.shape), _full(bi.shape), _full(br.shape),
        ],
        out_specs=pl.BlockSpec((blk, DW), lambda i: (i, 0)),
        out_shape=jax.ShapeDtypeStruct((NL, DW), jnp.float32),
    )(s, mx_, mn_, cnt2d, ls, w1, b1, w2, b2, w3, b3, wi, wr, bi, br)


def _tc_readout_body(ls, w1, b1, w2, b2, w3p, b3p, out):
    o = jnp.maximum(jnp.dot(ls[...][:, :D], w1[...],
                            preferred_element_type=jnp.float32) + b1[...], 0.0)
    o = jnp.maximum(jnp.dot(o, w2[...],
                            preferred_element_type=jnp.float32) + b2[...], 0.0)
    out[...] = jnp.dot(o, w3p[...],
                       preferred_element_type=jnp.float32) + b3p[...]


def _tc_readout(ls, w1, b1, w2, b2, w3p, b3p):
    blk = 2000
    return pl.pallas_call(
        _tc_readout_body,
        grid=(NL // blk,),
        in_specs=[
            pl.BlockSpec((blk, DW), lambda i: (i, 0)),
            _full(w1.shape), _full(b1.shape), _full(w2.shape), _full(b2.shape),
            _full(w3p.shape), _full(b3p.shape),
        ],
        out_specs=pl.BlockSpec((blk, 8), lambda i: (i, 0)),
        out_shape=jax.ShapeDtypeStruct((LPAD, 8), jnp.float32),
    )(ls, w1, b1, w2, b2, w3p, b3p)


def kernel(traffic, packets, capacity, scale, le_w1, le_b1, le_w2, le_b2,
           pe_w1, pe_b1, pe_w2, pe_b2, am_w1, am_b1, am_w2, am_b2, am_w3,
           am_b3, ro_w1, ro_b1, ro_w2, ro_b2, ro_w3, ro_b3, lg_wi, lg_wr,
           lg_bi, lg_br, pg_wi, pg_wr, pg_bi, pg_br, link_to_path, path_ids,
           sequence_path, sequence_links, n_links, n_paths):
    cs = jnp.stack([capacity, scale], axis=1)
    tp = jnp.stack([traffic, packets], axis=1)
    a_pad = jnp.pad(link_to_path, (0, NPP - NP_))
    cap_pad = jnp.pad(capacity, (0, LPAD - NL))
    scl_pad = jnp.pad(scale, (0, LPAD - NL))
    w3p = jnp.pad(ro_w3, ((0, 0), (0, 7)))
    b3p = jnp.pad(ro_b3, (0, 7))

    ls = _tc_link_embed(cs, le_w1, le_b1, le_w2, le_b2)
    plan_p, plan_l, nva, counts = _sc_plan(sequence_links)
    cnt2d = counts[:NL].reshape(NL, 1)

    for _ in range(8):
        g = _sc_gather(a_pad, ls)
        h = _tc_pgru(tp, g, pe_w1, pe_b1, pe_w2, pe_b2,
                     pg_wi, pg_wr, pg_bi, pg_br)
        s, mx_, mn_ = _sc_reduce(h, plan_p, plan_l, nva)
        ls = _tc_link(s, mx_, mn_, cnt2d, ls,
                      am_w1, am_b1, am_w2, am_b2, am_w3, am_b3,
                      lg_wi, lg_wr, lg_bi, lg_br)

    y = _tc_readout(ls, ro_w1, ro_b1, ro_w2, ro_b2, w3p, b3p)
    delay = _sc_delay(y, cap_pad, scl_pad)
    od = _sc_out_gather(delay, a_pad)
    return od[:NP_].reshape(NP_, 1)
